# baseline (speedup 1.0000x reference)
import functools

import jax
import jax.numpy as jnp
from jax import lax
from jax.experimental import pallas as pl
from jax.experimental.pallas import tpu as pltpu
from jax.experimental.pallas import tpu_sc as plsc

f32 = jnp.float32
i32 = jnp.int32

N = 10000
E = 320000
XD = 128
ED = 16
H = 256
QW = 64
L = 3
B = 64

NS = 16
NW = 2 * NS
EPT = E // NS
EPW = E // NW
C = 128
NCH1 = EPT // C
CT1 = EPT - NCH1 * C
NCH2 = EPW // C
CT2 = EPW - NCH2 * C
NODE_CH = N // C
NODE_CT = N - NODE_CH * C


def _leaky(v):
    return jnp.where(v >= 0, v, 0.01 * v)


def _enc_body(x_ref, w_ref, b_ref, att_ref, vidx_ref,
              h0_ref, h1_ref, h2_ref, h3_ref, alar_ref, hx_ref):
    h = lax.dot_general(x_ref[...], w_ref[...],
                        (((1,), (1,)), ((), ()))) + b_ref[...]
    h0_ref[...] = h[:, 0 * QW:1 * QW]
    h1_ref[...] = h[:, 1 * QW:2 * QW]
    h2_ref[...] = h[:, 2 * QW:3 * QW]
    h3_ref[...] = h[:, 3 * QW:4 * QW]
    alar_ref[...] = lax.dot_general(att_ref[...], h, (((1,), (1,)), ((), ())))
    ids = lax.broadcasted_iota(i32, (B, N), 1)
    oh = (ids == vidx_ref[...]).astype(f32)
    hx_ref[...] = lax.dot_general(oh, h, (((1,), (0,)), ((), ())))


def _tc_encode(x, enc_w, enc_b2, att8, vidx2):
    return pl.pallas_call(
        _enc_body,
        out_shape=(
            jax.ShapeDtypeStruct((N, QW), f32),
            jax.ShapeDtypeStruct((N, QW), f32),
            jax.ShapeDtypeStruct((N, QW), f32),
            jax.ShapeDtypeStruct((N, QW), f32),
            jax.ShapeDtypeStruct((8, N), f32),
            jax.ShapeDtypeStruct((B, H), f32),
        ),
    )(x, enc_w, enc_b2, att8, vidx2)


def _eenc_body(ea_ref, w_ref, b_ref, e0_ref, e1_ref, e2_ref, e3_ref):
    z = lax.dot_general(ea_ref[...], w_ref[...],
                        (((1,), (1,)), ((), ()))) + b_ref[...]
    e0_ref[...] = z[:, 0 * QW:1 * QW]
    e1_ref[...] = z[:, 1 * QW:2 * QW]
    e2_ref[...] = z[:, 2 * QW:3 * QW]
    e3_ref[...] = z[:, 3 * QW:4 * QW]


def _tc_eenc(edge_attr, eenc_w, eenc_b2):
    Te = 8000
    qspec = pl.BlockSpec((Te, QW), lambda i: (i, 0))
    return pl.pallas_call(
        _eenc_body,
        grid=(E // Te,),
        in_specs=[
            pl.BlockSpec((Te, ED), lambda i: (i, 0)),
            pl.BlockSpec((H, ED), lambda i: (0, 0)),
            pl.BlockSpec((1, H), lambda i: (0, 0)),
        ],
        out_specs=(qspec, qspec, qspec, qspec),
        out_shape=tuple(jax.ShapeDtypeStruct((E, QW), f32) for _ in range(4)),
    )(edge_attr, eenc_w, eenc_b2)


TROW = 2000


def _mlp1_body(a0_ref, a1_ref, a2_ref, a3_ref, w1_ref, b1_ref,
               z_ref, stats_ref):
    h2 = jnp.concatenate(
        [a0_ref[...], a1_ref[...], a2_ref[...], a3_ref[...]], axis=1)
    z = lax.dot_general(h2, w1_ref[...], (((1,), (1,)), ((), ()))) + b1_ref[...]
    z_ref[...] = z
    st = jnp.concatenate([jnp.sum(z, axis=0, keepdims=True),
                          jnp.sum(z * z, axis=0, keepdims=True)], axis=0)

    @pl.when(pl.program_id(0) == 0)
    def _():
        stats_ref[...] = st

    @pl.when(pl.program_id(0) > 0)
    def _():
        stats_ref[...] = stats_ref[...] + st


def _tc_mlp1(h2q, w1, b1_2):
    qspec = pl.BlockSpec((TROW, QW), lambda i: (i, 0))
    return pl.pallas_call(
        _mlp1_body,
        grid=(N // TROW,),
        in_specs=[qspec, qspec, qspec, qspec,
                  pl.BlockSpec((2 * H, H), lambda i: (0, 0)),
                  pl.BlockSpec((1, 2 * H), lambda i: (0, 0))],
        out_specs=(pl.BlockSpec((TROW, 2 * H), lambda i: (i, 0)),
                   pl.BlockSpec((2, 2 * H), lambda i: (0, 0))),
        out_shape=(jax.ShapeDtypeStruct((N, 2 * H), f32),
                   jax.ShapeDtypeStruct((2, 2 * H), f32)),
    )(*h2q, w1, b1_2)


def _mlp2_body(z_ref, stats_ref, i0_ref, i1_ref, i2_ref, i3_ref,
               w2_ref, b2_ref, g_ref, bb_ref, attn_ref, vidx_ref,
               h0_ref, h1_ref, h2_ref, h3_ref, alar_ref, sel_ref):
    mu = stats_ref[0:1, :] * (1.0 / N)
    var = stats_ref[1:2, :] * (1.0 / N) - mu * mu
    zn = _leaky((z_ref[...] - mu) * lax.rsqrt(var + 1e-5) * g_ref[...]
                + bb_ref[...])
    h2o = lax.dot_general(zn, w2_ref[...], (((1,), (1,)), ((), ()))) + b2_ref[...]
    pid = pl.program_id(0)
    ids = lax.broadcasted_iota(i32, (B, TROW), 1) + pid * TROW
    oh = (ids == vidx_ref[...]).astype(f32)
    selc = lax.dot_general(oh, h2o, (((1,), (0,)), ((), ())))

    @pl.when(pid == 0)
    def _():
        sel_ref[...] = selc

    @pl.when(pid > 0)
    def _():
        sel_ref[...] = sel_ref[...] + selc

    hn = h2o + jnp.concatenate(
        [i0_ref[...], i1_ref[...], i2_ref[...], i3_ref[...]], axis=1)
    h0_ref[...] = hn[:, 0 * QW:1 * QW]
    h1_ref[...] = hn[:, 1 * QW:2 * QW]
    h2_ref[...] = hn[:, 2 * QW:3 * QW]
    h3_ref[...] = hn[:, 3 * QW:4 * QW]
    alar_ref[...] = lax.dot_general(hn, attn_ref[...], (((1,), (1,)), ((), ())))


def _tc_mlp2(z, stats, idq, w2, b2_2, g2, bb2, att8n, vidx2):
    qspec = pl.BlockSpec((TROW, QW), lambda i: (i, 0))
    return pl.pallas_call(
        _mlp2_body,
        grid=(N // TROW,),
        in_specs=[pl.BlockSpec((TROW, 2 * H), lambda i: (i, 0)),
                  pl.BlockSpec((2, 2 * H), lambda i: (0, 0)),
                  qspec, qspec, qspec, qspec,
                  pl.BlockSpec((H, 2 * H), lambda i: (0, 0)),
                  pl.BlockSpec((1, H), lambda i: (0, 0)),
                  pl.BlockSpec((1, 2 * H), lambda i: (0, 0)),
                  pl.BlockSpec((1, 2 * H), lambda i: (0, 0)),
                  pl.BlockSpec((8, H), lambda i: (0, 0)),
                  pl.BlockSpec((B, 1), lambda i: (0, 0))],
        out_specs=(qspec, qspec, qspec, qspec,
                   pl.BlockSpec((TROW, 8), lambda i: (i, 0)),
                   pl.BlockSpec((B, H), lambda i: (0, 0))),
        out_shape=(jax.ShapeDtypeStruct((N, QW), f32),
                   jax.ShapeDtypeStruct((N, QW), f32),
                   jax.ShapeDtypeStruct((N, QW), f32),
                   jax.ShapeDtypeStruct((N, QW), f32),
                   jax.ShapeDtypeStruct((N, 8), f32),
                   jax.ShapeDtypeStruct((B, H), f32)),
    )(z, stats, *idq, w2, b2_2, g2, bb2, att8n, vidx2)


def _lstm_body(sel_ref, hx_ref, cx_ref, wih_ref, bih_ref, whh_ref, bhh_ref,
               fcw_ref, fcb_ref, hxo_ref, cxo_ref, out_ref):
    gates = (lax.dot_general(sel_ref[...], wih_ref[...], (((1,), (1,)), ((), ())))
             + bih_ref[...]
             + lax.dot_general(hx_ref[...], whh_ref[...], (((1,), (1,)), ((), ())))
             + bhh_ref[...])
    ig = jax.nn.sigmoid(gates[:, :H])
    fg = jax.nn.sigmoid(gates[:, H:2 * H])
    gg = jnp.tanh(gates[:, 2 * H:3 * H])
    og = jax.nn.sigmoid(gates[:, 3 * H:])
    cxn = fg * cx_ref[...] + ig * gg
    hxn = og * jnp.tanh(cxn)
    hxo_ref[...] = hxn
    cxo_ref[...] = cxn
    out_ref[...] = jax.nn.sigmoid(
        lax.dot_general(hxn, fcw_ref[...], (((1,), (1,)), ((), ())))
        + fcb_ref[...])


def _tc_lstm(sel, hx, cx, wih, bih2, whh, bhh2, fc8, fcb8):
    return pl.pallas_call(
        _lstm_body,
        out_shape=(jax.ShapeDtypeStruct((B, H), f32),
                   jax.ShapeDtypeStruct((B, H), f32),
                   jax.ShapeDtypeStruct((B, 8), f32)),
    )(sel, hx, cx, wih, bih2, whh, bhh2, fc8, fcb8)


COMB = 640


def _sc_alpha_body(srcH, dstH, alarH, alpha_out,
                   al_v, ar_v, den_v, src_v, dst_v, alpha_v, tmp_v, comb_v,
                   den_sh):
    c = lax.axis_index("c")
    s = lax.axis_index("s")
    zero16 = jnp.zeros((16,), f32)

    pltpu.sync_copy(alarH.at[pl.ds(0, N)], al_v)
    pltpu.sync_copy(alarH.at[pl.ds(N, N)], ar_v)

    def _zden(k, _):
        for u in range(5):
            den_v[pl.ds((k * 5 + u) * 16, 16)] = zero16
        return 0
    lax.fori_loop(0, N // 80, _zden, 0)

    pltpu.sync_copy(srcH.at[pl.ds(s * EPT, EPT)], src_v)
    pltpu.sync_copy(dstH.at[pl.ds(s * EPT, EPT)], dst_v)

    def _p1(g, _):
        for u in range(10):
            sl = pl.ds((g * 10 + u) * 16, 16)
            w = jnp.exp(_leaky(plsc.load_gather(al_v, [src_v[sl]])
                               + plsc.load_gather(ar_v, [dst_v[sl]])))
            plsc.addupdate_scatter(den_v, [dst_v[sl]], w)
        return 0
    lax.fori_loop(0, EPT // 160, _p1, 0)

    pltpu.sync_copy(den_v, den_sh.at[pl.ds(s * N, N)])
    plsc.subcore_barrier()

    def _comb(sz):
        off = s * COMB
        for q in range(sz // 16):
            comb_v[pl.ds(q * 16, 16)] = zero16

        def _addt(t, _):
            pltpu.sync_copy(den_sh.at[pl.ds(t * N + off, sz)],
                            tmp_v.at[pl.ds(0, sz)])
            for q in range(sz // 16):
                sl = pl.ds(q * 16, 16)
                comb_v[sl] = comb_v[sl] + tmp_v[sl]
            return 0
        lax.fori_loop(0, NS, _addt, 0)
        pltpu.sync_copy(comb_v.at[pl.ds(0, sz)],
                        den_sh.at[pl.ds(NS * N + off, sz)])

    @pl.when(s < NS - 1)
    def _():
        _comb(COMB)

    @pl.when(s == NS - 1)
    def _():
        _comb(N - (NS - 1) * COMB)

    plsc.subcore_barrier()
    pltpu.sync_copy(den_sh.at[pl.ds(NS * N, N)], den_v)

    w0 = (c * NS + s) * EPW
    pltpu.sync_copy(srcH.at[pl.ds(w0, EPW)], src_v.at[pl.ds(0, EPW)])
    pltpu.sync_copy(dstH.at[pl.ds(w0, EPW)], dst_v.at[pl.ds(0, EPW)])

    def _p2(g, _):
        for u in range(5):
            sl = pl.ds((g * 5 + u) * 16, 16)
            w = jnp.exp(_leaky(plsc.load_gather(al_v, [src_v[sl]])
                               + plsc.load_gather(ar_v, [dst_v[sl]])))
            dg = plsc.load_gather(den_v, [dst_v[sl]])
            alpha_v[sl] = w / (dg + 1e-16)
        return 0
    lax.fori_loop(0, EPW // 80, _p2, 0)
    pltpu.sync_copy(alpha_v, alpha_out.at[pl.ds(w0, EPW)])


def _sc_alpha(src, dst, alar2):
    mesh = plsc.VectorSubcoreMesh(core_axis_name="c", subcore_axis_name="s")
    fn = functools.partial(
        pl.kernel,
        mesh=mesh,
        compiler_params=pltpu.CompilerParams(needs_layout_passes=False),
        out_type=jax.ShapeDtypeStruct((E,), f32),
        scratch_types=[
            pltpu.VMEM((N,), f32),
            pltpu.VMEM((N,), f32),
            pltpu.VMEM((N,), f32),
            pltpu.VMEM((EPT,), i32),
            pltpu.VMEM((EPT,), i32),
            pltpu.VMEM((EPW,), f32),
            pltpu.VMEM((COMB,), f32),
            pltpu.VMEM((COMB,), f32),
            pltpu.VMEM_SHARED(((NS + 1) * N,), f32),
        ],
    )(_sc_alpha_body)
    return fn(src, dst, alar2)


RNG = 9984
NCHR = RNG // C
TE = EPT - 2 * RNG


def _sc_msg_body(hq0, hq1, eq0, eq1, alphaH, srcH, dstH,
                 oq0, oq1,
                 src_v, dst_v, alph_v, dstc0_v, dstc1_v, srct_v, dstt_v,
                 hr0_v, hr1_v, er0_v, er1_v, ms0_v, ms1_v,
                 acc_sh, sem0, sem1, sems0, sems1):
    c = lax.axis_index("c")
    s = lax.axis_index("s")
    e0 = s * EPT
    zero16 = jnp.zeros((16,), f32)

    def _zmsg(j, _):
        for q in range(QW // 16):
            ms0_v[j, pl.ds(q * 16, 16)] = zero16
        return 0
    lax.fori_loop(0, C, _zmsg, 0)

    def _zacc(k, _):
        ch = s + k * NS

        @pl.when(ch < NODE_CH)
        def _():
            pltpu.sync_copy(ms0_v, acc_sh.at[pl.ds(ch * C, C)])

        @pl.when(ch == NODE_CH)
        def _():
            pltpu.sync_copy(ms0_v.at[pl.ds(0, NODE_CT)],
                            acc_sh.at[pl.ds(NODE_CH * C, NODE_CT)])
        return 0
    lax.fori_loop(0, NODE_CH // NS + 1, _zacc, 0)
    plsc.subcore_barrier()

    bufs = ((dstc0_v, hr0_v, er0_v, ms0_v, sem0, sems0),
            (dstc1_v, hr1_v, er1_v, ms1_v, sem1, sems1))

    def _stage(h_ref, e_ref, r0, ci, b, wait_ok=True):
        dstc, hr, er, ms, sem, sems = bufs[b]
        off = ci * C
        if wait_ok:
            @pl.when(ci >= 2)
            def _():
                pltpu.make_async_copy(ms, acc_sh.at[dstc], sems).wait()
        for g in range(C // 16):
            sl = pl.ds(g * 16, 16)
            dstc[sl] = dst_v[pl.ds(off + g * 16, 16)]
        pltpu.async_copy(h_ref.at[src_v.at[pl.ds(off, C)]], hr, sem)
        pltpu.async_copy(e_ref.at[pl.ds(r0 + off, C)], er, sem)

    def _drain(h_ref, e_ref, b):
        dstc, hr, er, _, sem, _s = bufs[b]
        pltpu.make_async_copy(h_ref.at[src_v.at[pl.ds(0, C)]], hr, sem).wait()
        pltpu.make_async_copy(e_ref.at[pl.ds(e0, C)], er, sem).wait()

    U = 8

    def _compute(ci, b):
        dstc, hr, er, ms, _, sems = bufs[b]
        off = ci * C

        def _pe(j, _):
            for u in range(U):
                je = j * U + u
                av = plsc.load_gather(alph_v, [jnp.zeros((16,), i32) + off + je])
                for q in range(QW // 16):
                    qq = pl.ds(q * 16, 16)
                    t = hr[je, qq] + er[je, qq]
                    ms[je, qq] = av * jnp.maximum(t, 0.01 * t)
            return 0
        lax.fori_loop(0, C // U, _pe, 0)
        pltpu.async_copy(ms, acc_sh.at[dstc], sems, add=True)

    def _flush(b):
        dstc, _, _, ms, _, sems = bufs[b]
        pltpu.make_async_copy(ms, acc_sh.at[dstc], sems).wait()

    def _run_phase2(h_ref, e_ref):
        for r in range(2):
            r0 = e0 + r * RNG
            pltpu.sync_copy(srcH.at[pl.ds(r0, RNG)], src_v)
            pltpu.sync_copy(dstH.at[pl.ds(r0, RNG)], dst_v)
            pltpu.sync_copy(alphaH.at[pl.ds(r0, RNG)], alph_v)
            _stage(h_ref, e_ref, r0, 0, 0, wait_ok=False)

            def _outer(k, _):
                ci0 = k * 2
                for b in range(2):
                    ci = ci0 + b

                    @pl.when(ci + 1 < NCHR)
                    def _():
                        _stage(h_ref, e_ref, r0, ci + 1, (b + 1) % 2)

                    _drain(h_ref, e_ref, b)
                    _compute(ci, b)
                return 0
            lax.fori_loop(0, NCHR // 2, _outer, 0)
            _flush(0)
            _flush(1)

        toff = e0 + 2 * RNG
        pltpu.sync_copy(srcH.at[pl.ds(toff, TE)], srct_v)
        pltpu.sync_copy(dstH.at[pl.ds(toff, TE)], dstt_v)
        pltpu.sync_copy(alphaH.at[pl.ds(toff, TE)], alph_v.at[pl.ds(0, TE)])
        cpy = pltpu.async_copy(h_ref.at[srct_v], hr0_v.at[pl.ds(0, TE)], sem0)
        pltpu.sync_copy(e_ref.at[pl.ds(toff, TE)], er0_v.at[pl.ds(0, TE)])
        cpy.wait()

        def _pet(j, _):
            av = plsc.load_gather(alph_v, [jnp.zeros((16,), i32) + j])
            for q in range(QW // 16):
                qq = pl.ds(q * 16, 16)
                t = hr0_v[j, qq] + er0_v[j, qq]
                t = jnp.where(t >= 0., t, 0.01 * t)
                ms0_v[j, qq] = av * t
            return 0
        lax.fori_loop(0, TE, _pet, 0)
        pltpu.sync_copy(ms0_v.at[pl.ds(0, TE)], acc_sh.at[dstt_v], add=True)

    @pl.when(c == 0)
    def _():
        _run_phase2(hq0, eq0)

    @pl.when(c == 1)
    def _():
        _run_phase2(hq1, eq1)

    plsc.subcore_barrier()

    def _run_writeout(o_ref):
        def _w_chunk(k, _):
            ch = s + k * NS

            @pl.when(ch < NODE_CH)
            def _():
                pltpu.sync_copy(acc_sh.at[pl.ds(ch * C, C)],
                                o_ref.at[pl.ds(ch * C, C)])

            @pl.when(ch == NODE_CH)
            def _():
                pltpu.sync_copy(acc_sh.at[pl.ds(NODE_CH * C, NODE_CT)],
                                o_ref.at[pl.ds(NODE_CH * C, NODE_CT)])
            return 0
        lax.fori_loop(0, NODE_CH // NS + 1, _w_chunk, 0)

    @pl.when(c == 0)
    def _():
        _run_writeout(oq0)

    @pl.when(c == 1)
    def _():
        _run_writeout(oq1)


def _sc_msg(hq0, hq1, eq0, eq1, alpha, src, dst):
    mesh = plsc.VectorSubcoreMesh(core_axis_name="c", subcore_axis_name="s")
    fn = functools.partial(
        pl.kernel,
        mesh=mesh,
        compiler_params=pltpu.CompilerParams(
            needs_layout_passes=False, use_tc_tiling_on_sc=False),
        out_type=(
            jax.ShapeDtypeStruct((N, QW), f32),
            jax.ShapeDtypeStruct((N, QW), f32),
        ),
        scratch_types=[
            pltpu.VMEM((RNG,), i32),
            pltpu.VMEM((RNG,), i32),
            pltpu.VMEM((RNG,), f32),
            pltpu.VMEM((C,), i32),
            pltpu.VMEM((C,), i32),
            pltpu.VMEM((TE,), i32),
            pltpu.VMEM((TE,), i32),
            pltpu.VMEM((C, QW), f32),
            pltpu.VMEM((C, QW), f32),
            pltpu.VMEM((C, QW), f32),
            pltpu.VMEM((C, QW), f32),
            pltpu.VMEM((C, QW), f32),
            pltpu.VMEM((C, QW), f32),
            pltpu.VMEM_SHARED((N, QW), f32),
            pltpu.SemaphoreType.DMA,
            pltpu.SemaphoreType.DMA,
            pltpu.SemaphoreType.DMA,
            pltpu.SemaphoreType.DMA,
        ],
    )(_sc_msg_body)
    return fn(hq0, hq1, eq0, eq1, alpha, src, dst)


def kernel(x, edge_index, edge_attr, v_idx, enc_w, enc_b, eenc_w, eenc_b,
           att_l, att_r, mlp_w1, mlp_b1, bn_g, bn_b, mlp_w2, mlp_b2,
           lstm_wih, lstm_bih, lstm_whh, lstm_bhh, fc_w, fc_b):
    src = edge_index[0]
    dst = edge_index[1]
    vidx2 = v_idx.reshape(B, 1)
    enc_b2 = enc_b.reshape(1, H)
    eenc_b2 = eenc_b.reshape(1, H)
    bih2 = lstm_bih.reshape(1, 4 * H)
    bhh2 = lstm_bhh.reshape(1, 4 * H)
    fc8 = jnp.concatenate([fc_w, jnp.zeros((7, H), f32)], axis=0)
    fcb8 = jnp.concatenate([fc_b.reshape(1, 1), jnp.zeros((1, 7), f32)], axis=1)
    zeros6 = jnp.zeros((6, H), f32)
    att8 = [jnp.concatenate([att_l[i:i + 1], att_r[i:i + 1], zeros6], axis=0)
            for i in range(L)]
    att8.append(jnp.zeros((8, H), f32))

    hq = list(range(4))
    h0, h1, h2c, h3, alar, hx = _tc_encode(x, enc_w, enc_b2, att8[0], vidx2)
    hq = [h0, h1, h2c, h3]
    eq = list(_tc_eenc(edge_attr, eenc_w, eenc_b2))
    cx = jnp.zeros((B, H), f32)
    out = None
    for i in range(L):
        alpha = _sc_alpha(src, dst, alar.reshape(8 * N)[:2 * N])
        a0, a1 = _sc_msg(hq[0], hq[1], eq[0], eq[1], alpha, src, dst)
        a2, a3 = _sc_msg(hq[2], hq[3], eq[2], eq[3], alpha, src, dst)
        z, stats = _tc_mlp1([a0, a1, a2, a3], mlp_w1[i],
                            mlp_b1[i].reshape(1, 2 * H))
        h0, h1, h2c, h3, alarT, sel = _tc_mlp2(
            z, stats, hq, mlp_w2[i], mlp_b2[i].reshape(1, H),
            bn_g[i].reshape(1, 2 * H), bn_b[i].reshape(1, 2 * H),
            att8[i + 1], vidx2)
        alar = alarT.T
        hx, cx, out = _tc_lstm(sel, hx, cx, lstm_wih, bih2, lstm_whh, bhh2,
                               fc8, fcb8)
        hq = [h0, h1, h2c, h3]
    return out[:, :1]


# speedup vs baseline: 2.2753x; 2.2753x over previous
import functools

import jax
import jax.numpy as jnp
from jax import lax
from jax.experimental import pallas as pl
from jax.experimental.pallas import tpu as pltpu
from jax.experimental.pallas import tpu_sc as plsc

f32 = jnp.float32
i32 = jnp.int32

N = 10000
E = 320000
XD = 128
ED = 16
H = 256
QW = 64
L = 3
B = 64

NS = 16
NW = 2 * NS
EPT = E // NS
EPW = E // NW
C = 128
NCH1 = EPT // C
CT1 = EPT - NCH1 * C
NCH2 = EPW // C
CT2 = EPW - NCH2 * C
NODE_CH = N // C
NODE_CT = N - NODE_CH * C


def _leaky(v):
    return jnp.where(v >= 0, v, 0.01 * v)


def _enc_body(x_ref, w_ref, b_ref, att_ref, vidx_ref,
              h0_ref, h1_ref, h2_ref, h3_ref, alar_ref, hx_ref):
    h = lax.dot_general(x_ref[...], w_ref[...],
                        (((1,), (1,)), ((), ()))) + b_ref[...]
    h0_ref[...] = h[:, 0 * QW:1 * QW]
    h1_ref[...] = h[:, 1 * QW:2 * QW]
    h2_ref[...] = h[:, 2 * QW:3 * QW]
    h3_ref[...] = h[:, 3 * QW:4 * QW]
    alar_ref[...] = lax.dot_general(att_ref[...], h, (((1,), (1,)), ((), ())))
    ids = lax.broadcasted_iota(i32, (B, N), 1)
    oh = (ids == vidx_ref[...]).astype(f32)
    hx_ref[...] = lax.dot_general(oh, h, (((1,), (0,)), ((), ())))


def _tc_encode(x, enc_w, enc_b2, att8, vidx2):
    return pl.pallas_call(
        _enc_body,
        out_shape=(
            jax.ShapeDtypeStruct((N, QW), f32),
            jax.ShapeDtypeStruct((N, QW), f32),
            jax.ShapeDtypeStruct((N, QW), f32),
            jax.ShapeDtypeStruct((N, QW), f32),
            jax.ShapeDtypeStruct((8, N), f32),
            jax.ShapeDtypeStruct((B, H), f32),
        ),
    )(x, enc_w, enc_b2, att8, vidx2)


def _eenc_body(ea_ref, w_ref, b_ref, e0_ref, e1_ref, e2_ref, e3_ref):
    z = lax.dot_general(ea_ref[...], w_ref[...],
                        (((1,), (1,)), ((), ()))) + b_ref[...]
    e0_ref[...] = z[:, 0 * QW:1 * QW]
    e1_ref[...] = z[:, 1 * QW:2 * QW]
    e2_ref[...] = z[:, 2 * QW:3 * QW]
    e3_ref[...] = z[:, 3 * QW:4 * QW]


def _tc_eenc(edge_attr, eenc_w, eenc_b2):
    Te = 8000
    qspec = pl.BlockSpec((Te, QW), lambda i: (i, 0))
    return pl.pallas_call(
        _eenc_body,
        grid=(E // Te,),
        in_specs=[
            pl.BlockSpec((Te, ED), lambda i: (i, 0)),
            pl.BlockSpec((H, ED), lambda i: (0, 0)),
            pl.BlockSpec((1, H), lambda i: (0, 0)),
        ],
        out_specs=(qspec, qspec, qspec, qspec),
        out_shape=tuple(jax.ShapeDtypeStruct((E, QW), f32) for _ in range(4)),
    )(edge_attr, eenc_w, eenc_b2)


TROW = 2000


def _mlp1_body(a0_ref, a1_ref, a2_ref, a3_ref, w1_ref, b1_ref,
               z_ref, stats_ref):
    h2 = jnp.concatenate(
        [a0_ref[...], a1_ref[...], a2_ref[...], a3_ref[...]], axis=1)
    z = lax.dot_general(h2, w1_ref[...], (((1,), (1,)), ((), ()))) + b1_ref[...]
    z_ref[...] = z
    st = jnp.concatenate([jnp.sum(z, axis=0, keepdims=True),
                          jnp.sum(z * z, axis=0, keepdims=True)], axis=0)

    @pl.when(pl.program_id(0) == 0)
    def _():
        stats_ref[...] = st

    @pl.when(pl.program_id(0) > 0)
    def _():
        stats_ref[...] = stats_ref[...] + st


def _tc_mlp1(h2q, w1, b1_2):
    qspec = pl.BlockSpec((TROW, QW), lambda i: (i, 0))
    return pl.pallas_call(
        _mlp1_body,
        grid=(N // TROW,),
        in_specs=[qspec, qspec, qspec, qspec,
                  pl.BlockSpec((2 * H, H), lambda i: (0, 0)),
                  pl.BlockSpec((1, 2 * H), lambda i: (0, 0))],
        out_specs=(pl.BlockSpec((TROW, 2 * H), lambda i: (i, 0)),
                   pl.BlockSpec((2, 2 * H), lambda i: (0, 0))),
        out_shape=(jax.ShapeDtypeStruct((N, 2 * H), f32),
                   jax.ShapeDtypeStruct((2, 2 * H), f32)),
    )(*h2q, w1, b1_2)


def _mlp2_body(z_ref, stats_ref, i0_ref, i1_ref, i2_ref, i3_ref,
               w2_ref, b2_ref, g_ref, bb_ref, attn_ref, vidx_ref,
               h0_ref, h1_ref, h2_ref, h3_ref, alar_ref, sel_ref):
    mu = stats_ref[0:1, :] * (1.0 / N)
    var = stats_ref[1:2, :] * (1.0 / N) - mu * mu
    zn = _leaky((z_ref[...] - mu) * lax.rsqrt(var + 1e-5) * g_ref[...]
                + bb_ref[...])
    h2o = lax.dot_general(zn, w2_ref[...], (((1,), (1,)), ((), ()))) + b2_ref[...]
    pid = pl.program_id(0)
    ids = lax.broadcasted_iota(i32, (B, TROW), 1) + pid * TROW
    oh = (ids == vidx_ref[...]).astype(f32)
    selc = lax.dot_general(oh, h2o, (((1,), (0,)), ((), ())))

    @pl.when(pid == 0)
    def _():
        sel_ref[...] = selc

    @pl.when(pid > 0)
    def _():
        sel_ref[...] = sel_ref[...] + selc

    hn = h2o + jnp.concatenate(
        [i0_ref[...], i1_ref[...], i2_ref[...], i3_ref[...]], axis=1)
    h0_ref[...] = hn[:, 0 * QW:1 * QW]
    h1_ref[...] = hn[:, 1 * QW:2 * QW]
    h2_ref[...] = hn[:, 2 * QW:3 * QW]
    h3_ref[...] = hn[:, 3 * QW:4 * QW]
    alar_ref[...] = lax.dot_general(hn, attn_ref[...], (((1,), (1,)), ((), ())))


def _tc_mlp2(z, stats, idq, w2, b2_2, g2, bb2, att8n, vidx2):
    qspec = pl.BlockSpec((TROW, QW), lambda i: (i, 0))
    return pl.pallas_call(
        _mlp2_body,
        grid=(N // TROW,),
        in_specs=[pl.BlockSpec((TROW, 2 * H), lambda i: (i, 0)),
                  pl.BlockSpec((2, 2 * H), lambda i: (0, 0)),
                  qspec, qspec, qspec, qspec,
                  pl.BlockSpec((H, 2 * H), lambda i: (0, 0)),
                  pl.BlockSpec((1, H), lambda i: (0, 0)),
                  pl.BlockSpec((1, 2 * H), lambda i: (0, 0)),
                  pl.BlockSpec((1, 2 * H), lambda i: (0, 0)),
                  pl.BlockSpec((8, H), lambda i: (0, 0)),
                  pl.BlockSpec((B, 1), lambda i: (0, 0))],
        out_specs=(qspec, qspec, qspec, qspec,
                   pl.BlockSpec((TROW, 8), lambda i: (i, 0)),
                   pl.BlockSpec((B, H), lambda i: (0, 0))),
        out_shape=(jax.ShapeDtypeStruct((N, QW), f32),
                   jax.ShapeDtypeStruct((N, QW), f32),
                   jax.ShapeDtypeStruct((N, QW), f32),
                   jax.ShapeDtypeStruct((N, QW), f32),
                   jax.ShapeDtypeStruct((N, 8), f32),
                   jax.ShapeDtypeStruct((B, H), f32)),
    )(z, stats, *idq, w2, b2_2, g2, bb2, att8n, vidx2)


def _lstm_body(sel_ref, hx_ref, cx_ref, wih_ref, bih_ref, whh_ref, bhh_ref,
               fcw_ref, fcb_ref, hxo_ref, cxo_ref, out_ref):
    gates = (lax.dot_general(sel_ref[...], wih_ref[...], (((1,), (1,)), ((), ())))
             + bih_ref[...]
             + lax.dot_general(hx_ref[...], whh_ref[...], (((1,), (1,)), ((), ())))
             + bhh_ref[...])
    ig = jax.nn.sigmoid(gates[:, :H])
    fg = jax.nn.sigmoid(gates[:, H:2 * H])
    gg = jnp.tanh(gates[:, 2 * H:3 * H])
    og = jax.nn.sigmoid(gates[:, 3 * H:])
    cxn = fg * cx_ref[...] + ig * gg
    hxn = og * jnp.tanh(cxn)
    hxo_ref[...] = hxn
    cxo_ref[...] = cxn
    out_ref[...] = jax.nn.sigmoid(
        lax.dot_general(hxn, fcw_ref[...], (((1,), (1,)), ((), ())))
        + fcb_ref[...])


def _tc_lstm(sel, hx, cx, wih, bih2, whh, bhh2, fc8, fcb8):
    return pl.pallas_call(
        _lstm_body,
        out_shape=(jax.ShapeDtypeStruct((B, H), f32),
                   jax.ShapeDtypeStruct((B, H), f32),
                   jax.ShapeDtypeStruct((B, 8), f32)),
    )(sel, hx, cx, wih, bih2, whh, bhh2, fc8, fcb8)


COMB = 640


def _sc_alpha_body(srcH, dstH, alarH, alpha_out,
                   al_v, ar_v, den_v, src_v, dst_v, alpha_v, tmp_v, comb_v,
                   den_sh):
    c = lax.axis_index("c")
    s = lax.axis_index("s")
    zero16 = jnp.zeros((16,), f32)

    pltpu.sync_copy(alarH.at[pl.ds(0, N)], al_v)
    pltpu.sync_copy(alarH.at[pl.ds(N, N)], ar_v)

    def _zden(k, _):
        for u in range(5):
            den_v[pl.ds((k * 5 + u) * 16, 16)] = zero16
        return 0
    lax.fori_loop(0, N // 80, _zden, 0)

    pltpu.sync_copy(srcH.at[pl.ds(s * EPT, EPT)], src_v)
    pltpu.sync_copy(dstH.at[pl.ds(s * EPT, EPT)], dst_v)

    def _p1(g, _):
        for u in range(10):
            sl = pl.ds((g * 10 + u) * 16, 16)
            w = jnp.exp(_leaky(plsc.load_gather(al_v, [src_v[sl]])
                               + plsc.load_gather(ar_v, [dst_v[sl]])))
            plsc.addupdate_scatter(den_v, [dst_v[sl]], w)
        return 0
    lax.fori_loop(0, EPT // 160, _p1, 0)

    pltpu.sync_copy(den_v, den_sh.at[pl.ds(s * N, N)])
    plsc.subcore_barrier()

    def _comb(sz):
        off = s * COMB
        for q in range(sz // 16):
            comb_v[pl.ds(q * 16, 16)] = zero16

        def _addt(t, _):
            pltpu.sync_copy(den_sh.at[pl.ds(t * N + off, sz)],
                            tmp_v.at[pl.ds(0, sz)])
            for q in range(sz // 16):
                sl = pl.ds(q * 16, 16)
                comb_v[sl] = comb_v[sl] + tmp_v[sl]
            return 0
        lax.fori_loop(0, NS, _addt, 0)
        pltpu.sync_copy(comb_v.at[pl.ds(0, sz)],
                        den_sh.at[pl.ds(NS * N + off, sz)])

    @pl.when(s < NS - 1)
    def _():
        _comb(COMB)

    @pl.when(s == NS - 1)
    def _():
        _comb(N - (NS - 1) * COMB)

    plsc.subcore_barrier()
    pltpu.sync_copy(den_sh.at[pl.ds(NS * N, N)], den_v)

    w0 = (c * NS + s) * EPW
    pltpu.sync_copy(srcH.at[pl.ds(w0, EPW)], src_v.at[pl.ds(0, EPW)])
    pltpu.sync_copy(dstH.at[pl.ds(w0, EPW)], dst_v.at[pl.ds(0, EPW)])

    def _p2(g, _):
        for u in range(5):
            sl = pl.ds((g * 5 + u) * 16, 16)
            w = jnp.exp(_leaky(plsc.load_gather(al_v, [src_v[sl]])
                               + plsc.load_gather(ar_v, [dst_v[sl]])))
            dg = plsc.load_gather(den_v, [dst_v[sl]])
            alpha_v[sl] = w / (dg + 1e-16)
        return 0
    lax.fori_loop(0, EPW // 80, _p2, 0)
    pltpu.sync_copy(alpha_v, alpha_out.at[pl.ds(w0, EPW)])


def _sc_alpha(src, dst, alar2):
    mesh = plsc.VectorSubcoreMesh(core_axis_name="c", subcore_axis_name="s")
    fn = functools.partial(
        pl.kernel,
        mesh=mesh,
        compiler_params=pltpu.CompilerParams(needs_layout_passes=False),
        out_type=jax.ShapeDtypeStruct((E,), f32),
        scratch_types=[
            pltpu.VMEM((N,), f32),
            pltpu.VMEM((N,), f32),
            pltpu.VMEM((N,), f32),
            pltpu.VMEM((EPT,), i32),
            pltpu.VMEM((EPT,), i32),
            pltpu.VMEM((EPW,), f32),
            pltpu.VMEM((COMB,), f32),
            pltpu.VMEM((COMB,), f32),
            pltpu.VMEM_SHARED(((NS + 1) * N,), f32),
        ],
    )(_sc_alpha_body)
    return fn(src, dst, alar2)


RNG = 9984
NCHR = RNG // C
TE = EPT - 2 * RNG


def _sc_msg_body(hq0, hq1, eq0, eq1, alphaH, srcH, dstH,
                 oq0, oq1,
                 src_v, dst_v, alph_v, dstc0_v, dstc1_v, srct_v, dstt_v,
                 hr0_v, hr1_v, er0_v, er1_v, ms0_v, ms1_v,
                 acc_sh, sem0, sem1, sems0, sems1):
    c = lax.axis_index("c")
    s = lax.axis_index("s")
    e0 = s * EPT
    zero16 = jnp.zeros((16,), f32)

    def _zmsg(j, _):
        for q in range(QW // 16):
            ms0_v[j, pl.ds(q * 16, 16)] = zero16
        return 0
    lax.fori_loop(0, C, _zmsg, 0)

    def _zacc(k, _):
        ch = s + k * NS

        @pl.when(ch < NODE_CH)
        def _():
            pltpu.sync_copy(ms0_v, acc_sh.at[pl.ds(ch * C, C)])

        @pl.when(ch == NODE_CH)
        def _():
            pltpu.sync_copy(ms0_v.at[pl.ds(0, NODE_CT)],
                            acc_sh.at[pl.ds(NODE_CH * C, NODE_CT)])
        return 0
    lax.fori_loop(0, NODE_CH // NS + 1, _zacc, 0)
    plsc.subcore_barrier()

    bufs = ((dstc0_v, hr0_v, er0_v, ms0_v, sem0, sems0),
            (dstc1_v, hr1_v, er1_v, ms1_v, sem1, sems1))

    def _stage(h_ref, e_ref, r0, ci, b, wait_ok=True):
        dstc, hr, er, ms, sem, sems = bufs[b]
        off = ci * C
        if wait_ok:
            @pl.when(ci >= 2)
            def _():
                pltpu.make_async_copy(ms, acc_sh.at[dstc], sems).wait()
        for g in range(C // 16):
            sl = pl.ds(g * 16, 16)
            dstc[sl] = dst_v[pl.ds(off + g * 16, 16)]
        pltpu.async_copy(h_ref.at[src_v.at[pl.ds(off, C)]], hr, sem)
        pltpu.async_copy(e_ref.at[pl.ds(r0 + off, C)], er, sem)

    def _drain(h_ref, e_ref, b):
        dstc, hr, er, _, sem, _s = bufs[b]
        pltpu.make_async_copy(h_ref.at[src_v.at[pl.ds(0, C)]], hr, sem).wait()
        pltpu.make_async_copy(e_ref.at[pl.ds(e0, C)], er, sem).wait()

    U = 8

    def _compute(ci, b):
        dstc, hr, er, ms, _, sems = bufs[b]
        off = ci * C

        @plsc.parallel_loop(0, C, 1, unroll=U)
        def _pe(je):
            av = plsc.load_gather(alph_v, [jnp.zeros((16,), i32) + off + je])
            for q in range(QW // 16):
                qq = pl.ds(q * 16, 16)
                t = hr[je, qq] + er[je, qq]
                ms[je, qq] = av * jnp.maximum(t, 0.01 * t)
        pltpu.async_copy(ms, acc_sh.at[dstc], sems, add=True)

    def _flush(b):
        dstc, _, _, ms, _, sems = bufs[b]
        pltpu.make_async_copy(ms, acc_sh.at[dstc], sems).wait()

    def _run_phase2(h_ref, e_ref):
        for r in range(2):
            r0 = e0 + r * RNG
            pltpu.sync_copy(srcH.at[pl.ds(r0, RNG)], src_v)
            pltpu.sync_copy(dstH.at[pl.ds(r0, RNG)], dst_v)
            pltpu.sync_copy(alphaH.at[pl.ds(r0, RNG)], alph_v)
            _stage(h_ref, e_ref, r0, 0, 0, wait_ok=False)

            def _outer(k, _):
                ci0 = k * 2
                for b in range(2):
                    ci = ci0 + b

                    @pl.when(ci + 1 < NCHR)
                    def _():
                        _stage(h_ref, e_ref, r0, ci + 1, (b + 1) % 2)

                    _drain(h_ref, e_ref, b)
                    _compute(ci, b)
                return 0
            lax.fori_loop(0, NCHR // 2, _outer, 0)
            _flush(0)
            _flush(1)

        toff = e0 + 2 * RNG
        pltpu.sync_copy(srcH.at[pl.ds(toff, TE)], srct_v)
        pltpu.sync_copy(dstH.at[pl.ds(toff, TE)], dstt_v)
        pltpu.sync_copy(alphaH.at[pl.ds(toff, TE)], alph_v.at[pl.ds(0, TE)])
        cpy = pltpu.async_copy(h_ref.at[srct_v], hr0_v.at[pl.ds(0, TE)], sem0)
        pltpu.sync_copy(e_ref.at[pl.ds(toff, TE)], er0_v.at[pl.ds(0, TE)])
        cpy.wait()

        @plsc.parallel_loop(0, TE, 1, unroll=4)
        def _pet(j):
            av = plsc.load_gather(alph_v, [jnp.zeros((16,), i32) + j])
            for q in range(QW // 16):
                qq = pl.ds(q * 16, 16)
                t = hr0_v[j, qq] + er0_v[j, qq]
                ms0_v[j, qq] = av * jnp.maximum(t, 0.01 * t)
        pltpu.sync_copy(ms0_v.at[pl.ds(0, TE)], acc_sh.at[dstt_v], add=True)

    @pl.when(c == 0)
    def _():
        _run_phase2(hq0, eq0)

    @pl.when(c == 1)
    def _():
        _run_phase2(hq1, eq1)

    plsc.subcore_barrier()

    def _run_writeout(o_ref):
        def _w_chunk(k, _):
            ch = s + k * NS

            @pl.when(ch < NODE_CH)
            def _():
                pltpu.sync_copy(acc_sh.at[pl.ds(ch * C, C)],
                                o_ref.at[pl.ds(ch * C, C)])

            @pl.when(ch == NODE_CH)
            def _():
                pltpu.sync_copy(acc_sh.at[pl.ds(NODE_CH * C, NODE_CT)],
                                o_ref.at[pl.ds(NODE_CH * C, NODE_CT)])
            return 0
        lax.fori_loop(0, NODE_CH // NS + 1, _w_chunk, 0)

    @pl.when(c == 0)
    def _():
        _run_writeout(oq0)

    @pl.when(c == 1)
    def _():
        _run_writeout(oq1)


def _sc_msg(hq0, hq1, eq0, eq1, alpha, src, dst):
    mesh = plsc.VectorSubcoreMesh(core_axis_name="c", subcore_axis_name="s")
    fn = functools.partial(
        pl.kernel,
        mesh=mesh,
        compiler_params=pltpu.CompilerParams(
            needs_layout_passes=False, use_tc_tiling_on_sc=False),
        out_type=(
            jax.ShapeDtypeStruct((N, QW), f32),
            jax.ShapeDtypeStruct((N, QW), f32),
        ),
        scratch_types=[
            pltpu.VMEM((RNG,), i32),
            pltpu.VMEM((RNG,), i32),
            pltpu.VMEM((RNG,), f32),
            pltpu.VMEM((C,), i32),
            pltpu.VMEM((C,), i32),
            pltpu.VMEM((TE,), i32),
            pltpu.VMEM((TE,), i32),
            pltpu.VMEM((C, QW), f32),
            pltpu.VMEM((C, QW), f32),
            pltpu.VMEM((C, QW), f32),
            pltpu.VMEM((C, QW), f32),
            pltpu.VMEM((C, QW), f32),
            pltpu.VMEM((C, QW), f32),
            pltpu.VMEM_SHARED((N, QW), f32),
            pltpu.SemaphoreType.DMA,
            pltpu.SemaphoreType.DMA,
            pltpu.SemaphoreType.DMA,
            pltpu.SemaphoreType.DMA,
        ],
    )(_sc_msg_body)
    return fn(hq0, hq1, eq0, eq1, alpha, src, dst)


def kernel(x, edge_index, edge_attr, v_idx, enc_w, enc_b, eenc_w, eenc_b,
           att_l, att_r, mlp_w1, mlp_b1, bn_g, bn_b, mlp_w2, mlp_b2,
           lstm_wih, lstm_bih, lstm_whh, lstm_bhh, fc_w, fc_b):
    src = edge_index[0]
    dst = edge_index[1]
    vidx2 = v_idx.reshape(B, 1)
    enc_b2 = enc_b.reshape(1, H)
    eenc_b2 = eenc_b.reshape(1, H)
    bih2 = lstm_bih.reshape(1, 4 * H)
    bhh2 = lstm_bhh.reshape(1, 4 * H)
    fc8 = jnp.concatenate([fc_w, jnp.zeros((7, H), f32)], axis=0)
    fcb8 = jnp.concatenate([fc_b.reshape(1, 1), jnp.zeros((1, 7), f32)], axis=1)
    zeros6 = jnp.zeros((6, H), f32)
    att8 = [jnp.concatenate([att_l[i:i + 1], att_r[i:i + 1], zeros6], axis=0)
            for i in range(L)]
    att8.append(jnp.zeros((8, H), f32))

    hq = list(range(4))
    h0, h1, h2c, h3, alar, hx = _tc_encode(x, enc_w, enc_b2, att8[0], vidx2)
    hq = [h0, h1, h2c, h3]
    eq = list(_tc_eenc(edge_attr, eenc_w, eenc_b2))
    cx = jnp.zeros((B, H), f32)
    out = None
    for i in range(L):
        alpha = _sc_alpha(src, dst, alar.reshape(8 * N)[:2 * N])
        a0, a1 = _sc_msg(hq[0], hq[1], eq[0], eq[1], alpha, src, dst)
        a2, a3 = _sc_msg(hq[2], hq[3], eq[2], eq[3], alpha, src, dst)
        z, stats = _tc_mlp1([a0, a1, a2, a3], mlp_w1[i],
                            mlp_b1[i].reshape(1, 2 * H))
        h0, h1, h2c, h3, alarT, sel = _tc_mlp2(
            z, stats, hq, mlp_w2[i], mlp_b2[i].reshape(1, H),
            bn_g[i].reshape(1, 2 * H), bn_b[i].reshape(1, 2 * H),
            att8[i + 1], vidx2)
        alar = alarT.T
        hx, cx, out = _tc_lstm(sel, hx, cx, lstm_wih, bih2, lstm_whh, bhh2,
                               fc8, fcb8)
        hq = [h0, h1, h2c, h3]
    return out[:, :1]


# speedup vs baseline: 2.2927x; 1.0077x over previous
import functools

import jax
import jax.numpy as jnp
from jax import lax
from jax.experimental import pallas as pl
from jax.experimental.pallas import tpu as pltpu
from jax.experimental.pallas import tpu_sc as plsc

f32 = jnp.float32
i32 = jnp.int32

N = 10000
E = 320000
XD = 128
ED = 16
H = 256
QW = 64
L = 3
B = 64

NS = 16
NW = 2 * NS
EPT = E // NS
EPW = E // NW
C = 128
NCH1 = EPT // C
CT1 = EPT - NCH1 * C
NCH2 = EPW // C
CT2 = EPW - NCH2 * C
NODE_CH = N // C
NODE_CT = N - NODE_CH * C


def _leaky(v):
    return jnp.where(v >= 0, v, 0.01 * v)


def _enc_body(x_ref, w_ref, b_ref, att_ref, vidx_ref,
              h0_ref, h1_ref, h2_ref, h3_ref, alar_ref, hx_ref):
    h = lax.dot_general(x_ref[...], w_ref[...],
                        (((1,), (1,)), ((), ()))) + b_ref[...]
    h0_ref[...] = h[:, 0 * QW:1 * QW]
    h1_ref[...] = h[:, 1 * QW:2 * QW]
    h2_ref[...] = h[:, 2 * QW:3 * QW]
    h3_ref[...] = h[:, 3 * QW:4 * QW]
    alar_ref[...] = lax.dot_general(att_ref[...], h, (((1,), (1,)), ((), ())))
    ids = lax.broadcasted_iota(i32, (B, N), 1)
    oh = (ids == vidx_ref[...]).astype(f32)
    hx_ref[...] = lax.dot_general(oh, h, (((1,), (0,)), ((), ())))


def _tc_encode(x, enc_w, enc_b2, att8, vidx2):
    return pl.pallas_call(
        _enc_body,
        out_shape=(
            jax.ShapeDtypeStruct((N, QW), f32),
            jax.ShapeDtypeStruct((N, QW), f32),
            jax.ShapeDtypeStruct((N, QW), f32),
            jax.ShapeDtypeStruct((N, QW), f32),
            jax.ShapeDtypeStruct((8, N), f32),
            jax.ShapeDtypeStruct((B, H), f32),
        ),
    )(x, enc_w, enc_b2, att8, vidx2)


def _eenc_body(ea_ref, w_ref, b_ref, e0_ref, e1_ref, e2_ref, e3_ref):
    z = lax.dot_general(ea_ref[...], w_ref[...],
                        (((1,), (1,)), ((), ()))) + b_ref[...]
    e0_ref[...] = z[:, 0 * QW:1 * QW]
    e1_ref[...] = z[:, 1 * QW:2 * QW]
    e2_ref[...] = z[:, 2 * QW:3 * QW]
    e3_ref[...] = z[:, 3 * QW:4 * QW]


def _tc_eenc(edge_attr, eenc_w, eenc_b2):
    Te = 8000
    qspec = pl.BlockSpec((Te, QW), lambda i: (i, 0))
    return pl.pallas_call(
        _eenc_body,
        grid=(E // Te,),
        in_specs=[
            pl.BlockSpec((Te, ED), lambda i: (i, 0)),
            pl.BlockSpec((H, ED), lambda i: (0, 0)),
            pl.BlockSpec((1, H), lambda i: (0, 0)),
        ],
        out_specs=(qspec, qspec, qspec, qspec),
        out_shape=tuple(jax.ShapeDtypeStruct((E, QW), f32) for _ in range(4)),
    )(edge_attr, eenc_w, eenc_b2)


TROW = 2000


def _mlp1_body(a0_ref, a1_ref, a2_ref, a3_ref, w1_ref, b1_ref,
               z_ref, stats_ref):
    h2 = jnp.concatenate(
        [a0_ref[...], a1_ref[...], a2_ref[...], a3_ref[...]], axis=1)
    z = lax.dot_general(h2, w1_ref[...], (((1,), (1,)), ((), ()))) + b1_ref[...]
    z_ref[...] = z
    st = jnp.concatenate([jnp.sum(z, axis=0, keepdims=True),
                          jnp.sum(z * z, axis=0, keepdims=True)], axis=0)

    @pl.when(pl.program_id(0) == 0)
    def _():
        stats_ref[...] = st

    @pl.when(pl.program_id(0) > 0)
    def _():
        stats_ref[...] = stats_ref[...] + st


def _tc_mlp1(h2q, w1, b1_2):
    qspec = pl.BlockSpec((TROW, QW), lambda i: (i, 0))
    return pl.pallas_call(
        _mlp1_body,
        grid=(N // TROW,),
        in_specs=[qspec, qspec, qspec, qspec,
                  pl.BlockSpec((2 * H, H), lambda i: (0, 0)),
                  pl.BlockSpec((1, 2 * H), lambda i: (0, 0))],
        out_specs=(pl.BlockSpec((TROW, 2 * H), lambda i: (i, 0)),
                   pl.BlockSpec((2, 2 * H), lambda i: (0, 0))),
        out_shape=(jax.ShapeDtypeStruct((N, 2 * H), f32),
                   jax.ShapeDtypeStruct((2, 2 * H), f32)),
    )(*h2q, w1, b1_2)


def _mlp2_body(z_ref, stats_ref, i0_ref, i1_ref, i2_ref, i3_ref,
               w2_ref, b2_ref, g_ref, bb_ref, attn_ref, vidx_ref,
               h0_ref, h1_ref, h2_ref, h3_ref, alar_ref, sel_ref):
    mu = stats_ref[0:1, :] * (1.0 / N)
    var = stats_ref[1:2, :] * (1.0 / N) - mu * mu
    zn = _leaky((z_ref[...] - mu) * lax.rsqrt(var + 1e-5) * g_ref[...]
                + bb_ref[...])
    h2o = lax.dot_general(zn, w2_ref[...], (((1,), (1,)), ((), ()))) + b2_ref[...]
    pid = pl.program_id(0)
    ids = lax.broadcasted_iota(i32, (B, TROW), 1) + pid * TROW
    oh = (ids == vidx_ref[...]).astype(f32)
    selc = lax.dot_general(oh, h2o, (((1,), (0,)), ((), ())))

    @pl.when(pid == 0)
    def _():
        sel_ref[...] = selc

    @pl.when(pid > 0)
    def _():
        sel_ref[...] = sel_ref[...] + selc

    hn = h2o + jnp.concatenate(
        [i0_ref[...], i1_ref[...], i2_ref[...], i3_ref[...]], axis=1)
    h0_ref[...] = hn[:, 0 * QW:1 * QW]
    h1_ref[...] = hn[:, 1 * QW:2 * QW]
    h2_ref[...] = hn[:, 2 * QW:3 * QW]
    h3_ref[...] = hn[:, 3 * QW:4 * QW]
    alar_ref[...] = lax.dot_general(hn, attn_ref[...], (((1,), (1,)), ((), ())))


def _tc_mlp2(z, stats, idq, w2, b2_2, g2, bb2, att8n, vidx2):
    qspec = pl.BlockSpec((TROW, QW), lambda i: (i, 0))
    return pl.pallas_call(
        _mlp2_body,
        grid=(N // TROW,),
        in_specs=[pl.BlockSpec((TROW, 2 * H), lambda i: (i, 0)),
                  pl.BlockSpec((2, 2 * H), lambda i: (0, 0)),
                  qspec, qspec, qspec, qspec,
                  pl.BlockSpec((H, 2 * H), lambda i: (0, 0)),
                  pl.BlockSpec((1, H), lambda i: (0, 0)),
                  pl.BlockSpec((1, 2 * H), lambda i: (0, 0)),
                  pl.BlockSpec((1, 2 * H), lambda i: (0, 0)),
                  pl.BlockSpec((8, H), lambda i: (0, 0)),
                  pl.BlockSpec((B, 1), lambda i: (0, 0))],
        out_specs=(qspec, qspec, qspec, qspec,
                   pl.BlockSpec((TROW, 8), lambda i: (i, 0)),
                   pl.BlockSpec((B, H), lambda i: (0, 0))),
        out_shape=(jax.ShapeDtypeStruct((N, QW), f32),
                   jax.ShapeDtypeStruct((N, QW), f32),
                   jax.ShapeDtypeStruct((N, QW), f32),
                   jax.ShapeDtypeStruct((N, QW), f32),
                   jax.ShapeDtypeStruct((N, 8), f32),
                   jax.ShapeDtypeStruct((B, H), f32)),
    )(z, stats, *idq, w2, b2_2, g2, bb2, att8n, vidx2)


def _lstm_body(sel_ref, hx_ref, cx_ref, wih_ref, bih_ref, whh_ref, bhh_ref,
               fcw_ref, fcb_ref, hxo_ref, cxo_ref, out_ref):
    gates = (lax.dot_general(sel_ref[...], wih_ref[...], (((1,), (1,)), ((), ())))
             + bih_ref[...]
             + lax.dot_general(hx_ref[...], whh_ref[...], (((1,), (1,)), ((), ())))
             + bhh_ref[...])
    ig = jax.nn.sigmoid(gates[:, :H])
    fg = jax.nn.sigmoid(gates[:, H:2 * H])
    gg = jnp.tanh(gates[:, 2 * H:3 * H])
    og = jax.nn.sigmoid(gates[:, 3 * H:])
    cxn = fg * cx_ref[...] + ig * gg
    hxn = og * jnp.tanh(cxn)
    hxo_ref[...] = hxn
    cxo_ref[...] = cxn
    out_ref[...] = jax.nn.sigmoid(
        lax.dot_general(hxn, fcw_ref[...], (((1,), (1,)), ((), ())))
        + fcb_ref[...])


def _tc_lstm(sel, hx, cx, wih, bih2, whh, bhh2, fc8, fcb8):
    return pl.pallas_call(
        _lstm_body,
        out_shape=(jax.ShapeDtypeStruct((B, H), f32),
                   jax.ShapeDtypeStruct((B, H), f32),
                   jax.ShapeDtypeStruct((B, 8), f32)),
    )(sel, hx, cx, wih, bih2, whh, bhh2, fc8, fcb8)


COMB = 640


def _sc_alpha_body(srcH, dstH, alarH, alpha_out,
                   al_v, ar_v, den_v, src_v, dst_v, alpha_v, tmp_v, comb_v,
                   den_sh):
    c = lax.axis_index("c")
    s = lax.axis_index("s")
    zero16 = jnp.zeros((16,), f32)

    pltpu.sync_copy(alarH.at[pl.ds(0, N)], al_v)
    pltpu.sync_copy(alarH.at[pl.ds(N, N)], ar_v)

    def _zden(k, _):
        for u in range(5):
            den_v[pl.ds((k * 5 + u) * 16, 16)] = zero16
        return 0
    lax.fori_loop(0, N // 80, _zden, 0)

    pltpu.sync_copy(srcH.at[pl.ds(s * EPT, EPT)], src_v)
    pltpu.sync_copy(dstH.at[pl.ds(s * EPT, EPT)], dst_v)

    def _p1(g, _):
        for u in range(10):
            sl = pl.ds((g * 10 + u) * 16, 16)
            w = jnp.exp(_leaky(plsc.load_gather(al_v, [src_v[sl]])
                               + plsc.load_gather(ar_v, [dst_v[sl]])))
            plsc.addupdate_scatter(den_v, [dst_v[sl]], w)
        return 0
    lax.fori_loop(0, EPT // 160, _p1, 0)

    pltpu.sync_copy(den_v, den_sh.at[pl.ds(s * N, N)])
    plsc.subcore_barrier()

    def _comb(sz):
        off = s * COMB
        for q in range(sz // 16):
            comb_v[pl.ds(q * 16, 16)] = zero16

        def _addt(t, _):
            pltpu.sync_copy(den_sh.at[pl.ds(t * N + off, sz)],
                            tmp_v.at[pl.ds(0, sz)])
            for q in range(sz // 16):
                sl = pl.ds(q * 16, 16)
                comb_v[sl] = comb_v[sl] + tmp_v[sl]
            return 0
        lax.fori_loop(0, NS, _addt, 0)
        pltpu.sync_copy(comb_v.at[pl.ds(0, sz)],
                        den_sh.at[pl.ds(NS * N + off, sz)])

    @pl.when(s < NS - 1)
    def _():
        _comb(COMB)

    @pl.when(s == NS - 1)
    def _():
        _comb(N - (NS - 1) * COMB)

    plsc.subcore_barrier()
    pltpu.sync_copy(den_sh.at[pl.ds(NS * N, N)], den_v)

    w0 = (c * NS + s) * EPW
    pltpu.sync_copy(srcH.at[pl.ds(w0, EPW)], src_v.at[pl.ds(0, EPW)])
    pltpu.sync_copy(dstH.at[pl.ds(w0, EPW)], dst_v.at[pl.ds(0, EPW)])

    @plsc.parallel_loop(0, EPW // 16, 1, unroll=8)
    def _p2(g):
        sl = pl.ds(g * 16, 16)
        w = jnp.exp(_leaky(plsc.load_gather(al_v, [src_v[sl]])
                           + plsc.load_gather(ar_v, [dst_v[sl]])))
        dg = plsc.load_gather(den_v, [dst_v[sl]])
        alpha_v[sl] = w / (dg + 1e-16)
    pltpu.sync_copy(alpha_v, alpha_out.at[pl.ds(w0, EPW)])


def _sc_alpha(src, dst, alar2):
    mesh = plsc.VectorSubcoreMesh(core_axis_name="c", subcore_axis_name="s")
    fn = functools.partial(
        pl.kernel,
        mesh=mesh,
        compiler_params=pltpu.CompilerParams(needs_layout_passes=False),
        out_type=jax.ShapeDtypeStruct((E,), f32),
        scratch_types=[
            pltpu.VMEM((N,), f32),
            pltpu.VMEM((N,), f32),
            pltpu.VMEM((N,), f32),
            pltpu.VMEM((EPT,), i32),
            pltpu.VMEM((EPT,), i32),
            pltpu.VMEM((EPW,), f32),
            pltpu.VMEM((COMB,), f32),
            pltpu.VMEM((COMB,), f32),
            pltpu.VMEM_SHARED(((NS + 1) * N,), f32),
        ],
    )(_sc_alpha_body)
    return fn(src, dst, alar2)


RNG = 9984
NCHR = RNG // C
TE = EPT - 2 * RNG


def _sc_msg_body(hq0, hq1, eq0, eq1, alphaH, srcH, dstH,
                 oq0, oq1,
                 src_v, dst_v, alph_v, dstc0_v, dstc1_v, srct_v, dstt_v,
                 hr0_v, hr1_v, er0_v, er1_v, ms0_v, ms1_v,
                 acc_sh, sem0, sem1, sems0, sems1):
    c = lax.axis_index("c")
    s = lax.axis_index("s")
    e0 = s * EPT
    zero16 = jnp.zeros((16,), f32)

    def _zmsg(j, _):
        for q in range(QW // 16):
            ms0_v[j, pl.ds(q * 16, 16)] = zero16
        return 0
    lax.fori_loop(0, C, _zmsg, 0)

    def _zacc(k, _):
        ch = s + k * NS

        @pl.when(ch < NODE_CH)
        def _():
            pltpu.sync_copy(ms0_v, acc_sh.at[pl.ds(ch * C, C)])

        @pl.when(ch == NODE_CH)
        def _():
            pltpu.sync_copy(ms0_v.at[pl.ds(0, NODE_CT)],
                            acc_sh.at[pl.ds(NODE_CH * C, NODE_CT)])
        return 0
    lax.fori_loop(0, NODE_CH // NS + 1, _zacc, 0)
    plsc.subcore_barrier()

    bufs = ((dstc0_v, hr0_v, er0_v, ms0_v, sem0, sems0),
            (dstc1_v, hr1_v, er1_v, ms1_v, sem1, sems1))

    def _stage(h_ref, e_ref, r0, ci, b, wait_ok=True):
        dstc, hr, er, ms, sem, sems = bufs[b]
        off = ci * C
        if wait_ok:
            @pl.when(ci >= 2)
            def _():
                pltpu.make_async_copy(ms, acc_sh.at[dstc], sems).wait()
        for g in range(C // 16):
            sl = pl.ds(g * 16, 16)
            dstc[sl] = dst_v[pl.ds(off + g * 16, 16)]
        pltpu.async_copy(h_ref.at[src_v.at[pl.ds(off, C)]], hr, sem)
        pltpu.async_copy(e_ref.at[pl.ds(r0 + off, C)], er, sem)

    def _drain(h_ref, e_ref, b):
        dstc, hr, er, _, sem, _s = bufs[b]
        pltpu.make_async_copy(h_ref.at[src_v.at[pl.ds(0, C)]], hr, sem).wait()
        pltpu.make_async_copy(e_ref.at[pl.ds(e0, C)], er, sem).wait()

    U = 8

    def _compute(ci, b):
        dstc, hr, er, ms, _, sems = bufs[b]
        off = ci * C

        @plsc.parallel_loop(0, C, 1, unroll=U)
        def _pe(je):
            av = plsc.load_gather(alph_v, [jnp.zeros((16,), i32) + off + je])
            for q in range(QW // 16):
                qq = pl.ds(q * 16, 16)
                t = hr[je, qq] + er[je, qq]
                ms[je, qq] = av * jnp.maximum(t, 0.01 * t)
        pltpu.async_copy(ms, acc_sh.at[dstc], sems, add=True)

    def _flush(b):
        dstc, _, _, ms, _, sems = bufs[b]
        pltpu.make_async_copy(ms, acc_sh.at[dstc], sems).wait()

    def _run_phase2(h_ref, e_ref):
        for r in range(2):
            r0 = e0 + r * RNG
            pltpu.sync_copy(srcH.at[pl.ds(r0, RNG)], src_v)
            pltpu.sync_copy(dstH.at[pl.ds(r0, RNG)], dst_v)
            pltpu.sync_copy(alphaH.at[pl.ds(r0, RNG)], alph_v)
            _stage(h_ref, e_ref, r0, 0, 0, wait_ok=False)

            def _outer(k, _):
                ci0 = k * 2
                for b in range(2):
                    ci = ci0 + b

                    @pl.when(ci + 1 < NCHR)
                    def _():
                        _stage(h_ref, e_ref, r0, ci + 1, (b + 1) % 2)

                    _drain(h_ref, e_ref, b)
                    _compute(ci, b)
                return 0
            lax.fori_loop(0, NCHR // 2, _outer, 0)
            _flush(0)
            _flush(1)

        toff = e0 + 2 * RNG
        pltpu.sync_copy(srcH.at[pl.ds(toff, TE)], srct_v)
        pltpu.sync_copy(dstH.at[pl.ds(toff, TE)], dstt_v)
        pltpu.sync_copy(alphaH.at[pl.ds(toff, TE)], alph_v.at[pl.ds(0, TE)])
        cpy = pltpu.async_copy(h_ref.at[srct_v], hr0_v.at[pl.ds(0, TE)], sem0)
        pltpu.sync_copy(e_ref.at[pl.ds(toff, TE)], er0_v.at[pl.ds(0, TE)])
        cpy.wait()

        @plsc.parallel_loop(0, TE, 1, unroll=4)
        def _pet(j):
            av = plsc.load_gather(alph_v, [jnp.zeros((16,), i32) + j])
            for q in range(QW // 16):
                qq = pl.ds(q * 16, 16)
                t = hr0_v[j, qq] + er0_v[j, qq]
                ms0_v[j, qq] = av * jnp.maximum(t, 0.01 * t)
        pltpu.sync_copy(ms0_v.at[pl.ds(0, TE)], acc_sh.at[dstt_v], add=True)

    @pl.when(c == 0)
    def _():
        _run_phase2(hq0, eq0)

    @pl.when(c == 1)
    def _():
        _run_phase2(hq1, eq1)

    plsc.subcore_barrier()

    def _run_writeout(o_ref):
        def _w_chunk(k, _):
            ch = s + k * NS

            @pl.when(ch < NODE_CH)
            def _():
                pltpu.sync_copy(acc_sh.at[pl.ds(ch * C, C)],
                                o_ref.at[pl.ds(ch * C, C)])

            @pl.when(ch == NODE_CH)
            def _():
                pltpu.sync_copy(acc_sh.at[pl.ds(NODE_CH * C, NODE_CT)],
                                o_ref.at[pl.ds(NODE_CH * C, NODE_CT)])
            return 0
        lax.fori_loop(0, NODE_CH // NS + 1, _w_chunk, 0)

    @pl.when(c == 0)
    def _():
        _run_writeout(oq0)

    @pl.when(c == 1)
    def _():
        _run_writeout(oq1)


def _sc_msg(hq0, hq1, eq0, eq1, alpha, src, dst):
    mesh = plsc.VectorSubcoreMesh(core_axis_name="c", subcore_axis_name="s")
    fn = functools.partial(
        pl.kernel,
        mesh=mesh,
        compiler_params=pltpu.CompilerParams(
            needs_layout_passes=False, use_tc_tiling_on_sc=False),
        out_type=(
            jax.ShapeDtypeStruct((N, QW), f32),
            jax.ShapeDtypeStruct((N, QW), f32),
        ),
        scratch_types=[
            pltpu.VMEM((RNG,), i32),
            pltpu.VMEM((RNG,), i32),
            pltpu.VMEM((RNG,), f32),
            pltpu.VMEM((C,), i32),
            pltpu.VMEM((C,), i32),
            pltpu.VMEM((TE,), i32),
            pltpu.VMEM((TE,), i32),
            pltpu.VMEM((C, QW), f32),
            pltpu.VMEM((C, QW), f32),
            pltpu.VMEM((C, QW), f32),
            pltpu.VMEM((C, QW), f32),
            pltpu.VMEM((C, QW), f32),
            pltpu.VMEM((C, QW), f32),
            pltpu.VMEM_SHARED((N, QW), f32),
            pltpu.SemaphoreType.DMA,
            pltpu.SemaphoreType.DMA,
            pltpu.SemaphoreType.DMA,
            pltpu.SemaphoreType.DMA,
        ],
    )(_sc_msg_body)
    return fn(hq0, hq1, eq0, eq1, alpha, src, dst)


def kernel(x, edge_index, edge_attr, v_idx, enc_w, enc_b, eenc_w, eenc_b,
           att_l, att_r, mlp_w1, mlp_b1, bn_g, bn_b, mlp_w2, mlp_b2,
           lstm_wih, lstm_bih, lstm_whh, lstm_bhh, fc_w, fc_b):
    src = edge_index[0]
    dst = edge_index[1]
    vidx2 = v_idx.reshape(B, 1)
    enc_b2 = enc_b.reshape(1, H)
    eenc_b2 = eenc_b.reshape(1, H)
    bih2 = lstm_bih.reshape(1, 4 * H)
    bhh2 = lstm_bhh.reshape(1, 4 * H)
    fc8 = jnp.concatenate([fc_w, jnp.zeros((7, H), f32)], axis=0)
    fcb8 = jnp.concatenate([fc_b.reshape(1, 1), jnp.zeros((1, 7), f32)], axis=1)
    zeros6 = jnp.zeros((6, H), f32)
    att8 = [jnp.concatenate([att_l[i:i + 1], att_r[i:i + 1], zeros6], axis=0)
            for i in range(L)]
    att8.append(jnp.zeros((8, H), f32))

    hq = list(range(4))
    h0, h1, h2c, h3, alar, hx = _tc_encode(x, enc_w, enc_b2, att8[0], vidx2)
    hq = [h0, h1, h2c, h3]
    eq = list(_tc_eenc(edge_attr, eenc_w, eenc_b2))
    cx = jnp.zeros((B, H), f32)
    out = None
    for i in range(L):
        alpha = _sc_alpha(src, dst, alar.reshape(8 * N)[:2 * N])
        a0, a1 = _sc_msg(hq[0], hq[1], eq[0], eq[1], alpha, src, dst)
        a2, a3 = _sc_msg(hq[2], hq[3], eq[2], eq[3], alpha, src, dst)
        z, stats = _tc_mlp1([a0, a1, a2, a3], mlp_w1[i],
                            mlp_b1[i].reshape(1, 2 * H))
        h0, h1, h2c, h3, alarT, sel = _tc_mlp2(
            z, stats, hq, mlp_w2[i], mlp_b2[i].reshape(1, H),
            bn_g[i].reshape(1, 2 * H), bn_b[i].reshape(1, 2 * H),
            att8[i + 1], vidx2)
        alar = alarT.T
        hx, cx, out = _tc_lstm(sel, hx, cx, lstm_wih, bih2, lstm_whh, bhh2,
                               fc8, fcb8)
        hq = [h0, h1, h2c, h3]
    return out[:, :1]
